# baseline (speedup 1.0000x reference)
import functools

import jax
import jax.numpy as jnp
from jax import lax
from jax.experimental import pallas as pl
from jax.experimental.pallas import tpu as pltpu
from jax.experimental.pallas import tpu_sc as plsc

_N_TOK = 16384
_N_FEAT = 64
_N_EXPERTS = 16
_N_SEL = 4
_N_BINS = 5
_TEMPERATURE = 1.0

_NC = 2
_NS = 16
_NW = _NC * _NS
_TPW = _N_TOK // _NW
_L = 16
_GRP = _TPW // _L

_THR_ERF = (-0.8416212797164917, -0.25334709882736206,
            0.25334709882736206, 0.8416213393211365)
_THR_LIN = (0.20000000298023224, 0.4000000059604645,
            0.6000000238418579, 0.800000011920929)


def _sc_body(thr, x_hbm, idx_hbm, mask_hbm, cnt_hbm, bias_hbm,
             w_hbm, l_hbm, lo_hbm, hi_hbm,
             xv, idxv, maskv, cntv, biasv, wv, lv, lov, hiv):
    cid = lax.axis_index("c")
    sid = lax.axis_index("s")
    wid = sid * _NC + cid
    base = wid * _TPW

    pltpu.sync_copy(x_hbm.at[pl.ds(base * _N_FEAT, _TPW * _N_FEAT)], xv)
    pltpu.sync_copy(idx_hbm, idxv)
    pltpu.sync_copy(mask_hbm, maskv)
    pltpu.sync_copy(cnt_hbm, cntv)
    pltpu.sync_copy(bias_hbm, biasv)

    lanes = lax.broadcasted_iota(jnp.int32, (_L,), 0)
    one = jnp.full((_L,), 1.0, jnp.float32)
    zero = jnp.full((_L,), 0.0, jnp.float32)

    idx_vecs = [idxv[pl.ds(_L * q, _L)] for q in range(_N_EXPERTS * _N_SEL // _L)]
    mask_vecs = [maskv[pl.ds(_L * q, _L)] for q in range(_N_EXPERTS * _N_SEL // _L)]
    cnt_vec = cntv[...]
    bias_vec = biasv[...]
    col_sc = [idx_vecs[j // _L][j % _L] for j in range(_N_EXPERTS * _N_SEL)]
    mask_sc = [mask_vecs[j // _L][j % _L] for j in range(_N_EXPERTS * _N_SEL)]
    cnt_sc = [cnt_vec[e] for e in range(_N_EXPERTS)]
    bias_sc = [bias_vec[e] for e in range(_N_EXPERTS)]

    def group(g, carry):
        lo, hi = carry
        tok = g * _L + lanes

        logit_vecs = []
        for e in range(_N_EXPERTS):
            ts = []
            for s in range(_N_SEL):
                j = _N_SEL * e + s
                col = jnp.full((_L,), col_sc[j], jnp.int32)
                v = plsc.load_gather(xv, [tok * _N_FEAT + col])
                lo = jnp.minimum(lo, v)
                hi = jnp.maximum(hi, v)
                bins = (jnp.where(v >= thr[0], one, zero)
                        + jnp.where(v >= thr[1], one, zero)
                        + jnp.where(v >= thr[2], one, zero)
                        + jnp.where(v >= thr[3], one, zero))
                bc = (bins + 0.5) / float(_N_BINS)
                ts.append(bc * mask_sc[j])
            acc = (ts[0] + ts[2]) + (ts[1] + ts[3])
            le = acc / cnt_sc[e] + bias_sc[e]
            logit_vecs.append(le)
            plsc.store_scatter(
                lv, [tok * _N_EXPERTS + jnp.full((_L,), e, jnp.int32)], le)

        m1 = logit_vecs[0]
        i1 = jnp.full((_L,), 0, jnp.int32)
        m2 = jnp.full((_L,), -jnp.inf, jnp.float32)
        i2 = jnp.full((_L,), 0, jnp.int32)
        for e in range(1, _N_EXPERTS):
            le = logit_vecs[e]
            esp = jnp.full((_L,), e, jnp.int32)
            gt1 = le > m1
            gt2 = le > m2
            m2 = jnp.where(gt1, m1, jnp.where(gt2, le, m2))
            i2 = jnp.where(gt1, i1, jnp.where(gt2, esp, i2))
            m1 = jnp.where(gt1, le, m1)
            i1 = jnp.where(gt1, esp, i1)

        b = jnp.exp(m2 - m1)
        w1 = 1.0 / (1.0 + b)
        w2 = b / (1.0 + b)
        for e in range(_N_EXPERTS):
            esp = jnp.full((_L,), e, jnp.int32)
            we = (jnp.where(i1 == esp, w1, zero)
                  + jnp.where(i2 == esp, w2, zero))
            plsc.store_scatter(wv, [tok * _N_EXPERTS + esp], we)
        return lo, hi

    lo0 = jnp.full((_L,), jnp.inf, jnp.float32)
    hi0 = jnp.full((_L,), -jnp.inf, jnp.float32)
    lo, hi = lax.fori_loop(0, _GRP, group, (lo0, hi0))

    lov[...] = lo
    hiv[...] = hi
    pltpu.sync_copy(lov, lo_hbm.at[pl.ds(wid * _L, _L)])
    pltpu.sync_copy(hiv, hi_hbm.at[pl.ds(wid * _L, _L)])
    pltpu.sync_copy(wv, w_hbm.at[pl.ds(base * _N_EXPERTS, _TPW * _N_EXPERTS)])
    pltpu.sync_copy(lv, l_hbm.at[pl.ds(base * _N_EXPERTS, _TPW * _N_EXPERTS)])


def _make_sc_call(thr):
    mesh = plsc.VectorSubcoreMesh(core_axis_name="c", subcore_axis_name="s")
    out_type = (
        jax.ShapeDtypeStruct((_N_TOK * _N_EXPERTS,), jnp.float32),
        jax.ShapeDtypeStruct((_N_TOK * _N_EXPERTS,), jnp.float32),
        jax.ShapeDtypeStruct((_NW * _L,), jnp.float32),
        jax.ShapeDtypeStruct((_NW * _L,), jnp.float32),
    )
    scratch = [
        pltpu.VMEM((_TPW * _N_FEAT,), jnp.float32),
        pltpu.VMEM((_N_EXPERTS * _N_SEL,), jnp.int32),
        pltpu.VMEM((_N_EXPERTS * _N_SEL,), jnp.float32),
        pltpu.VMEM((_N_EXPERTS,), jnp.float32),
        pltpu.VMEM((_N_EXPERTS,), jnp.float32),
        pltpu.VMEM((_TPW * _N_EXPERTS,), jnp.float32),
        pltpu.VMEM((_TPW * _N_EXPERTS,), jnp.float32),
        pltpu.VMEM((_L,), jnp.float32),
        pltpu.VMEM((_L,), jnp.float32),
    ]
    return pl.kernel(
        functools.partial(_sc_body, thr),
        out_type=out_type,
        mesh=mesh,
        scratch_types=scratch,
        compiler_params=pltpu.CompilerParams(needs_layout_passes=False),
    )


def kernel(rule_features, selected_mask, expert_bias, selected_idx):
    x = rule_features.astype(jnp.float32)
    idx_flat = selected_idx.reshape(-1).astype(jnp.int32)
    mask_flat = selected_mask.astype(jnp.float32).reshape(-1)
    count = jnp.maximum(jnp.sum(selected_mask, axis=-1), 1.0)
    count = count.astype(jnp.float32)
    bias = expert_bias.astype(jnp.float32)

    args = (x.reshape(-1), idx_flat, mask_flat, count, bias)
    w_erf, l_erf, lo_part, hi_part = _make_sc_call(_THR_ERF)(*args)
    w_erf = w_erf.reshape(_N_TOK, _N_EXPERTS)
    l_erf = l_erf.reshape(_N_TOK, _N_EXPERTS)
    lo = jnp.min(lo_part)
    hi = jnp.max(hi_part)
    already = jnp.logical_and(lo >= -1e-06, hi <= 1.0 + 1e-06)

    def _lin_path(operands):
        w, l, _, _ = _make_sc_call(_THR_LIN)(*operands)
        return (w.reshape(_N_TOK, _N_EXPERTS), l.reshape(_N_TOK, _N_EXPERTS))

    def _erf_path(_):
        return (w_erf, l_erf)

    weights, scaled_logits = jax.lax.cond(already, _lin_path, _erf_path, args)
    return (weights, scaled_logits)


# speedup vs baseline: 1.0943x; 1.0943x over previous
import functools

import jax
import jax.numpy as jnp
from jax import lax
from jax.experimental import pallas as pl
from jax.experimental.pallas import tpu as pltpu
from jax.experimental.pallas import tpu_sc as plsc

_N_TOK = 16384
_N_FEAT = 64
_N_EXPERTS = 16
_N_SEL = 4
_N_BINS = 5
_TEMPERATURE = 1.0

_NC = 2
_NS = 16
_NW = _NC * _NS
_TPW = _N_TOK // _NW
_L = 16
_GRP = _TPW // _L
_XS = 65
_OS = 17

_THR_ERF = (-0.8416212797164917, -0.25334709882736206,
            0.25334709882736206, 0.8416213393211365)
_THR_LIN = (0.20000000298023224, 0.4000000059604645,
            0.6000000238418579, 0.800000011920929)


def _sc_body(thr, x_hbm, idx_hbm, mask_hbm, cnt_hbm, bias_hbm,
             w_hbm, l_hbm, lo_hbm, hi_hbm,
             xvp, idxv, maskv, cntv, biasv, wv, lv, wvp, lvp, lov, hiv):
    cid = lax.axis_index("c")
    sid = lax.axis_index("s")
    wid = sid * _NC + cid
    base = wid * _TPW

    pltpu.sync_copy(x_hbm.at[pl.ds(base * _N_FEAT, _TPW * _N_FEAT)],
                    xvp.at[pl.ds(0, _TPW * _N_FEAT)])
    pltpu.sync_copy(idx_hbm, idxv)
    pltpu.sync_copy(mask_hbm, maskv)
    pltpu.sync_copy(cnt_hbm, cntv)
    pltpu.sync_copy(bias_hbm, biasv)

    lanes = lax.broadcasted_iota(jnp.int32, (_L,), 0)
    one = jnp.full((_L,), 1.0, jnp.float32)
    zero = jnp.full((_L,), 0.0, jnp.float32)
    zero_i = jnp.full((_L,), 0, jnp.int32)

    def repack(i, c):
        t = _TPW - 1 - i
        r0 = xvp[pl.ds(t * _N_FEAT, _L)]
        r1 = xvp[pl.ds(t * _N_FEAT + _L, _L)]
        r2 = xvp[pl.ds(t * _N_FEAT + 2 * _L, _L)]
        r3 = xvp[pl.ds(t * _N_FEAT + 3 * _L, _L)]
        xvp[pl.ds(t * _XS, _L)] = r0
        xvp[pl.ds(t * _XS + _L, _L)] = r1
        xvp[pl.ds(t * _XS + 2 * _L, _L)] = r2
        xvp[pl.ds(t * _XS + 3 * _L, _L)] = r3
        return c

    lax.fori_loop(0, _TPW, repack, 0, unroll=4)

    idx_vecs = [idxv[pl.ds(_L * q, _L)] for q in range(_N_EXPERTS * _N_SEL // _L)]
    mask_vecs = [maskv[pl.ds(_L * q, _L)] for q in range(_N_EXPERTS * _N_SEL // _L)]
    cnt_vec = cntv[...]
    bias_vec = biasv[...]
    col_sc = [idx_vecs[j // _L][j % _L] for j in range(_N_EXPERTS * _N_SEL)]
    mask_sc = [mask_vecs[j // _L][j % _L] for j in range(_N_EXPERTS * _N_SEL)]
    cnt_sc = [cnt_vec[e] for e in range(_N_EXPERTS)]
    bias_sc = [bias_vec[e] for e in range(_N_EXPERTS)]

    colv = [jnp.full((_L,), col_sc[j], jnp.int32)
            for j in range(_N_EXPERTS * _N_SEL)]
    espv = [jnp.full((_L,), e, jnp.int32) for e in range(_N_EXPERTS)]
    lanes_xs = lanes * _XS
    lanes_os = lanes * _OS

    def group(g, carry):
        lo, hi = carry
        tok_xs = lanes_xs + jnp.full((_L,), g * (_L * _XS), jnp.int32)
        tok_os = lanes_os + jnp.full((_L,), g * (_L * _OS), jnp.int32)

        m1 = zero
        i1 = zero_i
        m2 = zero
        i2 = zero_i
        for e in range(_N_EXPERTS):
            ts = []
            for s in range(_N_SEL):
                j = _N_SEL * e + s
                v = plsc.load_gather(xvp, [tok_xs + colv[j]])
                lo = jnp.minimum(lo, v)
                hi = jnp.maximum(hi, v)
                bins = (jnp.where(v >= thr[0], one, zero)
                        + jnp.where(v >= thr[1], one, zero)
                        + jnp.where(v >= thr[2], one, zero)
                        + jnp.where(v >= thr[3], one, zero))
                bc = (bins + 0.5) / float(_N_BINS)
                ts.append(bc * mask_sc[j])
            acc = (ts[0] + ts[2]) + (ts[1] + ts[3])
            le = acc / cnt_sc[e] + bias_sc[e]
            plsc.store_scatter(lvp, [tok_os + espv[e]], le)
            if e == 0:
                m1 = le
                m2 = jnp.full((_L,), -jnp.inf, jnp.float32)
            else:
                gt1 = le > m1
                gt2 = le > m2
                m2 = jnp.where(gt1, m1, jnp.where(gt2, le, m2))
                i2 = jnp.where(gt1, i1, jnp.where(gt2, espv[e], i2))
                m1 = jnp.where(gt1, le, m1)
                i1 = jnp.where(gt1, espv[e], i1)

        b = jnp.exp(m2 - m1)
        w1 = 1.0 / (1.0 + b)
        w2 = b / (1.0 + b)
        for e in range(_N_EXPERTS):
            we = (jnp.where(i1 == espv[e], w1, zero)
                  + jnp.where(i2 == espv[e], w2, zero))
            plsc.store_scatter(wvp, [tok_os + espv[e]], we)
        return lo, hi

    lo0 = jnp.full((_L,), jnp.inf, jnp.float32)
    hi0 = jnp.full((_L,), -jnp.inf, jnp.float32)
    lo, hi = lax.fori_loop(0, _GRP, group, (lo0, hi0))

    def compact(i, c):
        wrow = wvp[pl.ds(i * _OS, _L)]
        lrow = lvp[pl.ds(i * _OS, _L)]
        wv[pl.ds(i * _N_EXPERTS, _L)] = wrow
        lv[pl.ds(i * _N_EXPERTS, _L)] = lrow
        return c

    lax.fori_loop(0, _TPW, compact, 0, unroll=4)

    lov[...] = lo
    hiv[...] = hi
    pltpu.sync_copy(lov, lo_hbm.at[pl.ds(wid * _L, _L)])
    pltpu.sync_copy(hiv, hi_hbm.at[pl.ds(wid * _L, _L)])
    pltpu.sync_copy(wv, w_hbm.at[pl.ds(base * _N_EXPERTS, _TPW * _N_EXPERTS)])
    pltpu.sync_copy(lv, l_hbm.at[pl.ds(base * _N_EXPERTS, _TPW * _N_EXPERTS)])


def _make_sc_call(thr):
    mesh = plsc.VectorSubcoreMesh(core_axis_name="c", subcore_axis_name="s")
    out_type = (
        jax.ShapeDtypeStruct((_N_TOK * _N_EXPERTS,), jnp.float32),
        jax.ShapeDtypeStruct((_N_TOK * _N_EXPERTS,), jnp.float32),
        jax.ShapeDtypeStruct((_NW * _L,), jnp.float32),
        jax.ShapeDtypeStruct((_NW * _L,), jnp.float32),
    )
    scratch = [
        pltpu.VMEM((_TPW * _XS,), jnp.float32),
        pltpu.VMEM((_N_EXPERTS * _N_SEL,), jnp.int32),
        pltpu.VMEM((_N_EXPERTS * _N_SEL,), jnp.float32),
        pltpu.VMEM((_N_EXPERTS,), jnp.float32),
        pltpu.VMEM((_N_EXPERTS,), jnp.float32),
        pltpu.VMEM((_TPW * _N_EXPERTS,), jnp.float32),
        pltpu.VMEM((_TPW * _N_EXPERTS,), jnp.float32),
        pltpu.VMEM((_TPW * _OS,), jnp.float32),
        pltpu.VMEM((_TPW * _OS,), jnp.float32),
        pltpu.VMEM((_L,), jnp.float32),
        pltpu.VMEM((_L,), jnp.float32),
    ]
    return pl.kernel(
        functools.partial(_sc_body, thr),
        out_type=out_type,
        mesh=mesh,
        scratch_types=scratch,
        compiler_params=pltpu.CompilerParams(needs_layout_passes=False),
    )


def kernel(rule_features, selected_mask, expert_bias, selected_idx):
    x = rule_features.astype(jnp.float32)
    idx_flat = selected_idx.reshape(-1).astype(jnp.int32)
    mask_flat = selected_mask.astype(jnp.float32).reshape(-1)
    count = jnp.maximum(jnp.sum(selected_mask, axis=-1), 1.0)
    count = count.astype(jnp.float32)
    bias = expert_bias.astype(jnp.float32)

    args = (x.reshape(-1), idx_flat, mask_flat, count, bias)
    w_erf, l_erf, lo_part, hi_part = _make_sc_call(_THR_ERF)(*args)
    w_erf = w_erf.reshape(_N_TOK, _N_EXPERTS)
    l_erf = l_erf.reshape(_N_TOK, _N_EXPERTS)
    lo = jnp.min(lo_part)
    hi = jnp.max(hi_part)
    already = jnp.logical_and(lo >= -1e-06, hi <= 1.0 + 1e-06)

    def _lin_path(operands):
        w, l, _, _ = _make_sc_call(_THR_LIN)(*operands)
        return (w.reshape(_N_TOK, _N_EXPERTS), l.reshape(_N_TOK, _N_EXPERTS))

    def _erf_path(_):
        return (w_erf, l_erf)

    weights, scaled_logits = jax.lax.cond(already, _lin_path, _erf_path, args)
    return (weights, scaled_logits)


# speedup vs baseline: 1.1472x; 1.0484x over previous
import functools

import jax
import jax.numpy as jnp
from jax import lax
from jax.experimental import pallas as pl
from jax.experimental.pallas import tpu as pltpu
from jax.experimental.pallas import tpu_sc as plsc

_N_TOK = 16384
_N_FEAT = 64
_N_EXPERTS = 16
_N_SEL = 4
_N_BINS = 5
_TEMPERATURE = 1.0

_NC = 2
_NS = 16
_NW = _NC * _NS
_TPW = _N_TOK // _NW
_L = 16
_GRP = _TPW // _L
_XS = 65
_OS = 17

_THR_ERF = (-0.8416212797164917, -0.25334709882736206,
            0.25334709882736206, 0.8416213393211365)
_THR_LIN = (0.20000000298023224, 0.4000000059604645,
            0.6000000238418579, 0.800000011920929)


def _sc_body(thr, x_hbm, idx_hbm, mask_hbm, cnt_hbm, bias_hbm,
             w_hbm, l_hbm, lo_hbm, hi_hbm,
             xvp, idxv, maskv, cntv, biasv, wv, lv, wvp, lvp, lov, hiv):
    cid = lax.axis_index("c")
    sid = lax.axis_index("s")
    wid = sid * _NC + cid
    base = wid * _TPW

    pltpu.sync_copy(x_hbm.at[pl.ds(base * _N_FEAT, _TPW * _N_FEAT)],
                    xvp.at[pl.ds(0, _TPW * _N_FEAT)])
    pltpu.sync_copy(idx_hbm, idxv)
    pltpu.sync_copy(mask_hbm, maskv)
    pltpu.sync_copy(cnt_hbm, cntv)
    pltpu.sync_copy(bias_hbm, biasv)

    lanes = lax.broadcasted_iota(jnp.int32, (_L,), 0)
    one = jnp.full((_L,), 1.0, jnp.float32)
    zero = jnp.full((_L,), 0.0, jnp.float32)
    zero_i = jnp.full((_L,), 0, jnp.int32)

    def repack(i, c):
        t = _TPW - 1 - i
        r0 = xvp[pl.ds(t * _N_FEAT, _L)]
        r1 = xvp[pl.ds(t * _N_FEAT + _L, _L)]
        r2 = xvp[pl.ds(t * _N_FEAT + 2 * _L, _L)]
        r3 = xvp[pl.ds(t * _N_FEAT + 3 * _L, _L)]
        xvp[pl.ds(t * _XS, _L)] = r0
        xvp[pl.ds(t * _XS + _L, _L)] = r1
        xvp[pl.ds(t * _XS + 2 * _L, _L)] = r2
        xvp[pl.ds(t * _XS + 3 * _L, _L)] = r3
        return c

    lax.fori_loop(0, _TPW, repack, 0, unroll=4)

    idx_vecs = [idxv[pl.ds(_L * q, _L)] for q in range(_N_EXPERTS * _N_SEL // _L)]
    mask_vecs = [maskv[pl.ds(_L * q, _L)] for q in range(_N_EXPERTS * _N_SEL // _L)]
    cnt_vec = cntv[...]
    bias_vec = biasv[...]
    col_sc = [idx_vecs[j // _L][j % _L] for j in range(_N_EXPERTS * _N_SEL)]
    mask_sc = [mask_vecs[j // _L][j % _L] for j in range(_N_EXPERTS * _N_SEL)]
    cnt_sc = [cnt_vec[e] for e in range(_N_EXPERTS)]
    bias_sc = [bias_vec[e] for e in range(_N_EXPERTS)]

    colv = [jnp.full((_L,), col_sc[j], jnp.int32)
            for j in range(_N_EXPERTS * _N_SEL)]
    espv = [jnp.full((_L,), e, jnp.int32) for e in range(_N_EXPERTS)]
    lanes_xs = lanes * _XS
    lanes_os = lanes * _OS
    bcv = [jnp.full((_L,), c, jnp.float32) for c in (
        0.10000000149011612, 0.30000001192092896, 0.5,
        0.699999988079071, 0.8999999761581421)]

    def group(g, carry):
        lo, hi = carry
        tok_xs = lanes_xs + jnp.full((_L,), g * (_L * _XS), jnp.int32)
        tok_os = lanes_os + jnp.full((_L,), g * (_L * _OS), jnp.int32)

        m1 = zero
        i1 = zero_i
        m2 = zero
        i2 = zero_i
        for e in range(_N_EXPERTS):
            ts = []
            for s in range(_N_SEL):
                j = _N_SEL * e + s
                v = plsc.load_gather(xvp, [tok_xs + colv[j]])
                lo = jnp.minimum(lo, v)
                hi = jnp.maximum(hi, v)
                bc = jnp.where(
                    v >= thr[1],
                    jnp.where(v >= thr[2],
                              jnp.where(v >= thr[3], bcv[4], bcv[3]), bcv[2]),
                    jnp.where(v >= thr[0], bcv[1], bcv[0]))
                ts.append(bc * mask_sc[j])
            acc = (ts[0] + ts[2]) + (ts[1] + ts[3])
            le = acc / cnt_sc[e] + bias_sc[e]
            plsc.store_scatter(lvp, [tok_os + espv[e]], le)
            if e == 0:
                m1 = le
                m2 = jnp.full((_L,), -jnp.inf, jnp.float32)
            else:
                gt1 = le > m1
                gt2 = le > m2
                m2 = jnp.where(gt1, m1, jnp.where(gt2, le, m2))
                i2 = jnp.where(gt1, i1, jnp.where(gt2, espv[e], i2))
                m1 = jnp.where(gt1, le, m1)
                i1 = jnp.where(gt1, espv[e], i1)

        b = jnp.exp(m2 - m1)
        w1 = 1.0 / (1.0 + b)
        w2 = b / (1.0 + b)
        for e in range(_N_EXPERTS):
            we = jnp.where(i1 == espv[e], w1,
                           jnp.where(i2 == espv[e], w2, zero))
            plsc.store_scatter(wvp, [tok_os + espv[e]], we)
        return lo, hi

    lo0 = jnp.full((_L,), jnp.inf, jnp.float32)
    hi0 = jnp.full((_L,), -jnp.inf, jnp.float32)
    lo, hi = lax.fori_loop(0, _GRP, group, (lo0, hi0))

    def compact(i, c):
        wrow = wvp[pl.ds(i * _OS, _L)]
        lrow = lvp[pl.ds(i * _OS, _L)]
        wv[pl.ds(i * _N_EXPERTS, _L)] = wrow
        lv[pl.ds(i * _N_EXPERTS, _L)] = lrow
        return c

    lax.fori_loop(0, _TPW, compact, 0, unroll=4)

    lov[...] = lo
    hiv[...] = hi
    pltpu.sync_copy(lov, lo_hbm.at[pl.ds(wid * _L, _L)])
    pltpu.sync_copy(hiv, hi_hbm.at[pl.ds(wid * _L, _L)])
    pltpu.sync_copy(wv, w_hbm.at[pl.ds(base * _N_EXPERTS, _TPW * _N_EXPERTS)])
    pltpu.sync_copy(lv, l_hbm.at[pl.ds(base * _N_EXPERTS, _TPW * _N_EXPERTS)])


def _make_sc_call(thr):
    mesh = plsc.VectorSubcoreMesh(core_axis_name="c", subcore_axis_name="s")
    out_type = (
        jax.ShapeDtypeStruct((_N_TOK * _N_EXPERTS,), jnp.float32),
        jax.ShapeDtypeStruct((_N_TOK * _N_EXPERTS,), jnp.float32),
        jax.ShapeDtypeStruct((_NW * _L,), jnp.float32),
        jax.ShapeDtypeStruct((_NW * _L,), jnp.float32),
    )
    scratch = [
        pltpu.VMEM((_TPW * _XS,), jnp.float32),
        pltpu.VMEM((_N_EXPERTS * _N_SEL,), jnp.int32),
        pltpu.VMEM((_N_EXPERTS * _N_SEL,), jnp.float32),
        pltpu.VMEM((_N_EXPERTS,), jnp.float32),
        pltpu.VMEM((_N_EXPERTS,), jnp.float32),
        pltpu.VMEM((_TPW * _N_EXPERTS,), jnp.float32),
        pltpu.VMEM((_TPW * _N_EXPERTS,), jnp.float32),
        pltpu.VMEM((_TPW * _OS,), jnp.float32),
        pltpu.VMEM((_TPW * _OS,), jnp.float32),
        pltpu.VMEM((_L,), jnp.float32),
        pltpu.VMEM((_L,), jnp.float32),
    ]
    return pl.kernel(
        functools.partial(_sc_body, thr),
        out_type=out_type,
        mesh=mesh,
        scratch_types=scratch,
        compiler_params=pltpu.CompilerParams(needs_layout_passes=False),
    )


def kernel(rule_features, selected_mask, expert_bias, selected_idx):
    x = rule_features.astype(jnp.float32)
    idx_flat = selected_idx.reshape(-1).astype(jnp.int32)
    mask_flat = selected_mask.astype(jnp.float32).reshape(-1)
    count = jnp.maximum(jnp.sum(selected_mask, axis=-1), 1.0)
    count = count.astype(jnp.float32)
    bias = expert_bias.astype(jnp.float32)

    args = (x.reshape(-1), idx_flat, mask_flat, count, bias)
    w_erf, l_erf, lo_part, hi_part = _make_sc_call(_THR_ERF)(*args)
    w_erf = w_erf.reshape(_N_TOK, _N_EXPERTS)
    l_erf = l_erf.reshape(_N_TOK, _N_EXPERTS)
    lo = jnp.min(lo_part)
    hi = jnp.max(hi_part)
    already = jnp.logical_and(lo >= -1e-06, hi <= 1.0 + 1e-06)

    def _lin_path(operands):
        w, l, _, _ = _make_sc_call(_THR_LIN)(*operands)
        return (w.reshape(_N_TOK, _N_EXPERTS), l.reshape(_N_TOK, _N_EXPERTS))

    def _erf_path(_):
        return (w_erf, l_erf)

    weights, scaled_logits = jax.lax.cond(already, _lin_path, _erf_path, args)
    return (weights, scaled_logits)
